# Initial kernel scaffold; baseline (speedup 1.0000x reference)
import jax
import jax.numpy as jnp
from jax.experimental import pallas as pl


def kernel(x, codebook):
    raise NotImplementedError("write your pallas kernel here")


# speedup vs baseline: 3.9214x; 3.9214x over previous
import functools

import jax
import jax.numpy as jnp
from jax.experimental import pallas as pl
from jax.experimental.pallas import tpu as pltpu

_K = 512
_D = 32
_H = 128
_W = 128


def _vq_kernel(x_ref, cb_ref, codes_ref, idx_ref, *, tile):
    xb = x_ref[0]
    cb = cb_ref[...]
    scores = jax.lax.dot_general(
        cb, xb, (((1,), (0,)), ((), ())),
        preferred_element_type=jnp.float32)
    cb_sqr = jnp.sum(cb * cb, axis=1)
    dist = cb_sqr[:, None] - 2.0 * scores
    minval = jnp.min(dist, axis=0, keepdims=True)
    kiota = jax.lax.broadcasted_iota(jnp.int32, dist.shape, 0)
    idx = jnp.min(jnp.where(dist == minval, kiota, _K), axis=0)
    onehot = (kiota == idx[None, :]).astype(jnp.float32)
    codes = jax.lax.dot_general(
        cb, onehot, (((0,), (0,)), ((), ())),
        preferred_element_type=jnp.float32)
    codes_ref[0] = codes
    idx_ref[0] = idx.reshape(tile // _W, _W)


def kernel(x, codebook):
    B, D, H, W = x.shape
    N = H * W
    tile = 2048
    rows = tile // W
    xf = x.reshape(B, D, N)
    grid = (B, N // tile)
    codes, idx = pl.pallas_call(
        functools.partial(_vq_kernel, tile=tile),
        grid=grid,
        in_specs=[
            pl.BlockSpec((1, D, tile), lambda b, t: (b, 0, t)),
            pl.BlockSpec((_K, D), lambda b, t: (0, 0)),
        ],
        out_specs=[
            pl.BlockSpec((1, D, tile), lambda b, t: (b, 0, t)),
            pl.BlockSpec((1, rows, W), lambda b, t: (b, t, 0)),
        ],
        out_shape=[
            jax.ShapeDtypeStruct((B, D, N), jnp.float32),
            jax.ShapeDtypeStruct((B, H, W), jnp.int32),
        ],
    )(xf, codebook)
    return codes.reshape(B, D, H, W), idx


# speedup vs baseline: 4.7960x; 1.2230x over previous
import functools

import jax
import jax.numpy as jnp
from jax.experimental import pallas as pl

_K = 512
_W = 128


def _vq_kernel(x_ref, cb_ref, gm_ref, codes_ref, idx_ref, *, tile):
    xb = x_ref[0]
    cb = cb_ref[...]
    scores = jax.lax.dot_general(
        cb, xb, (((1,), (0,)), ((), ())),
        preferred_element_type=jnp.float32)
    half_sqr = 0.5 * jnp.sum(cb * cb, axis=1)
    scores = scores - half_sqr[:, None]
    maxval = jnp.max(scores, axis=0, keepdims=True)
    mask = (scores == maxval).astype(jnp.float32)
    out = jax.lax.dot_general(
        gm_ref[...], mask, (((0,), (0,)), ((), ())),
        preferred_element_type=jnp.float32)
    codes_ref[0] = out[:-1]
    idx_ref[0] = out[-1].astype(jnp.int32).reshape(tile // _W, _W)


def kernel(x, codebook):
    B, D, H, W = x.shape
    N = H * W
    tile = 2048
    rows = tile // W
    xf = x.reshape(B, D, N)
    kvec = jax.lax.iota(jnp.float32, _K).reshape(_K, 1)
    gm = jnp.concatenate([codebook, kvec], axis=1)
    grid = (B, N // tile)
    codes, idx = pl.pallas_call(
        functools.partial(_vq_kernel, tile=tile),
        grid=grid,
        in_specs=[
            pl.BlockSpec((1, D, tile), lambda b, t: (b, 0, t)),
            pl.BlockSpec((_K, D), lambda b, t: (0, 0)),
            pl.BlockSpec((_K, D + 1), lambda b, t: (0, 0)),
        ],
        out_specs=[
            pl.BlockSpec((1, D, tile), lambda b, t: (b, 0, t)),
            pl.BlockSpec((1, rows, W), lambda b, t: (b, t, 0)),
        ],
        out_shape=[
            jax.ShapeDtypeStruct((B, D, N), jnp.float32),
            jax.ShapeDtypeStruct((B, H, W), jnp.int32),
        ],
    )(xf, codebook, gm)
    return codes.reshape(B, D, H, W), idx


# speedup vs baseline: 7.4195x; 1.5470x over previous
import functools

import jax
import jax.numpy as jnp
from jax.experimental import pallas as pl

_K = 512
_W = 128


def _vq_kernel(x_ref, cb_ref, gm_ref, codes_ref, idx_ref, *, tile):
    D = x_ref.shape[1]
    xb = x_ref[0].reshape(D, tile)
    cb = cb_ref[...]
    scores = jax.lax.dot_general(
        cb, xb, (((1,), (0,)), ((), ())),
        preferred_element_type=jnp.float32)
    half_sqr = 0.5 * jnp.sum(cb * cb, axis=1)
    scores = scores - half_sqr[:, None]
    maxval = jnp.max(scores, axis=0, keepdims=True)
    mask = (scores == maxval).astype(jnp.float32)
    out = jax.lax.dot_general(
        gm_ref[...], mask, (((0,), (0,)), ((), ())),
        preferred_element_type=jnp.float32)
    codes_ref[0] = out[:-1].reshape(D, tile // _W, _W)
    idx_ref[0] = out[-1].astype(jnp.int32).reshape(tile // _W, _W)


def kernel(x, codebook):
    B, D, H, W = x.shape
    N = H * W
    tile = 2048
    rows = tile // W
    kvec = jax.lax.iota(jnp.float32, _K).reshape(_K, 1)
    gm = jnp.concatenate([codebook, kvec], axis=1)
    grid = (B, H // rows)
    codes, idx = pl.pallas_call(
        functools.partial(_vq_kernel, tile=tile),
        grid=grid,
        in_specs=[
            pl.BlockSpec((1, D, rows, W), lambda b, t: (b, 0, t, 0)),
            pl.BlockSpec((_K, D), lambda b, t: (0, 0)),
            pl.BlockSpec((_K, D + 1), lambda b, t: (0, 0)),
        ],
        out_specs=[
            pl.BlockSpec((1, D, rows, W), lambda b, t: (b, 0, t, 0)),
            pl.BlockSpec((1, rows, W), lambda b, t: (b, t, 0)),
        ],
        out_shape=[
            jax.ShapeDtypeStruct((B, D, H, W), jnp.float32),
            jax.ShapeDtypeStruct((B, H, W), jnp.int32),
        ],
    )(x, codebook, gm)
    return codes, idx


# speedup vs baseline: 7.7808x; 1.0487x over previous
import functools

import jax
import jax.numpy as jnp
from jax.experimental import pallas as pl

_K = 512
_W = 128


def _vq_kernel(x_ref, cb_ref, gm_ref, codes_ref, idx_ref, *, tile):
    D = x_ref.shape[1]
    xb = x_ref[0].reshape(D, tile)
    cb = cb_ref[...]
    scores = jax.lax.dot_general(
        cb, xb, (((1,), (0,)), ((), ())),
        preferred_element_type=jnp.float32)
    half_sqr = 0.5 * jnp.sum(cb * cb, axis=1)
    scores = scores - half_sqr[:, None]
    maxval = jnp.max(scores, axis=0, keepdims=True)
    mask = (scores == maxval).astype(jnp.float32)
    out = jax.lax.dot_general(
        gm_ref[...], mask, (((0,), (0,)), ((), ())),
        preferred_element_type=jnp.float32)
    codes_ref[0] = out[:-1].reshape(D, tile // _W, _W)
    idx_ref[0] = out[-1].astype(jnp.int32).reshape(tile // _W, _W)


def kernel(x, codebook):
    B, D, H, W = x.shape
    N = H * W
    tile = 4096
    rows = tile // W
    kvec = jax.lax.iota(jnp.float32, _K).reshape(_K, 1)
    gm = jnp.concatenate([codebook, kvec], axis=1)
    grid = (B, H // rows)
    codes, idx = pl.pallas_call(
        functools.partial(_vq_kernel, tile=tile),
        grid=grid,
        in_specs=[
            pl.BlockSpec((1, D, rows, W), lambda b, t: (b, 0, t, 0)),
            pl.BlockSpec((_K, D), lambda b, t: (0, 0)),
            pl.BlockSpec((_K, D + 1), lambda b, t: (0, 0)),
        ],
        out_specs=[
            pl.BlockSpec((1, D, rows, W), lambda b, t: (b, 0, t, 0)),
            pl.BlockSpec((1, rows, W), lambda b, t: (b, t, 0)),
        ],
        out_shape=[
            jax.ShapeDtypeStruct((B, D, H, W), jnp.float32),
            jax.ShapeDtypeStruct((B, H, W), jnp.int32),
        ],
    )(x, codebook, gm)
    return codes, idx


# speedup vs baseline: 8.0945x; 1.0403x over previous
import functools

import jax
import jax.numpy as jnp
from jax.experimental import pallas as pl

_K = 512
_W = 128


def _vq_kernel(x_ref, cb_ref, gm_ref, codes_ref, idx_ref, *, tile):
    D = x_ref.shape[1]
    xb = x_ref[0].reshape(D, tile)
    cb = cb_ref[...]
    scores = jax.lax.dot_general(
        cb, xb, (((1,), (0,)), ((), ())),
        preferred_element_type=jnp.float32)
    half_sqr = 0.5 * jnp.sum(cb * cb, axis=1)
    scores = scores - half_sqr[:, None]
    maxval = jnp.max(scores, axis=0, keepdims=True)
    mask = (scores == maxval).astype(jnp.float32)
    out = jax.lax.dot_general(
        gm_ref[...], mask, (((0,), (0,)), ((), ())),
        preferred_element_type=jnp.float32)
    codes_ref[0] = out[:-1].reshape(D, tile // _W, _W)
    idx_ref[0] = out[-1].astype(jnp.int32).reshape(tile // _W, _W)


def kernel(x, codebook):
    B, D, H, W = x.shape
    N = H * W
    tile = 8192
    rows = tile // W
    kvec = jax.lax.iota(jnp.float32, _K).reshape(_K, 1)
    gm = jnp.concatenate([codebook, kvec], axis=1)
    grid = (B, H // rows)
    codes, idx = pl.pallas_call(
        functools.partial(_vq_kernel, tile=tile),
        grid=grid,
        in_specs=[
            pl.BlockSpec((1, D, rows, W), lambda b, t: (b, 0, t, 0)),
            pl.BlockSpec((_K, D), lambda b, t: (0, 0)),
            pl.BlockSpec((_K, D + 1), lambda b, t: (0, 0)),
        ],
        out_specs=[
            pl.BlockSpec((1, D, rows, W), lambda b, t: (b, 0, t, 0)),
            pl.BlockSpec((1, rows, W), lambda b, t: (b, t, 0)),
        ],
        out_shape=[
            jax.ShapeDtypeStruct((B, D, H, W), jnp.float32),
            jax.ShapeDtypeStruct((B, H, W), jnp.int32),
        ],
    )(x, codebook, gm)
    return codes, idx
